# baseline (speedup 1.0000x reference)
import functools

import jax
import jax.numpy as jnp
from jax import lax
from jax.experimental import pallas as pl
from jax.experimental.pallas import tpu as pltpu
from jax.experimental.pallas import tpu_sc as plsc

B = 1024
V = 100000
NC = 2
NS = 16
L = 16
NW = NC * NS
PER = B // NW

_mesh = plsc.VectorSubcoreMesh(core_axis_name="c", subcore_axis_name="s")


@functools.partial(
    pl.kernel,
    out_type=jax.ShapeDtypeStruct((NC * L,), jnp.float32),
    mesh=_mesh,
    compiler_params=pltpu.CompilerParams(
        needs_layout_passes=False, use_tc_tiling_on_sc=True),
    scratch_types=[
        pltpu.VMEM((PER,), jnp.int32),
        pltpu.VMEM((PER, 128), jnp.float32),
        pltpu.VMEM((PER,), jnp.float32),
        pltpu.VMEM((NS * L,), jnp.float32),
        pltpu.VMEM((L,), jnp.float32),
        pltpu.VMEM_SHARED((NS * L,), jnp.float32),
        pltpu.SemaphoreType.DMA,
    ],
)
def _nll_sc(xt_hbm, tgt_hbm, out_hbm, idx_v, rows_v, diag_v, buf_v, out_v,
            shared, sem):
    cid = lax.axis_index("c")
    sid = lax.axis_index("s")
    wid = sid * NC + cid
    base = wid * PER

    cb = (wid // 4) * 128
    pltpu.sync_copy(tgt_hbm.at[pl.ds(base, PER)], idx_v)
    pltpu.async_copy(xt_hbm.at[idx_v, pl.ds(cb, 128)], rows_v, sem).wait()

    part = None
    for j in range(PER // L):
        rid = j * L + lax.iota(jnp.int32, L)
        cid_col = (wid % 4) * PER + rid
        vals = plsc.load_gather(rows_v, [rid, cid_col])
        part = vals if part is None else part + vals
    diag_v[pl.ds(0, L)] = part
    pltpu.sync_copy(diag_v.at[pl.ds(0, L)], shared.at[pl.ds(sid * L, L)])

    plsc.subcore_barrier()

    @pl.when(sid == 0)
    def _():
        pltpu.sync_copy(shared, buf_v)
        acc = buf_v[pl.ds(0, L)]
        for r in range(1, NS):
            acc = acc + buf_v[pl.ds(r * L, L)]
        out_v[...] = plsc.cumsum(acc * (-1.0 / B))
        pltpu.sync_copy(out_v, out_hbm.at[pl.ds(cid * L, L)])


def kernel(input_tensor, target_tensor):
    out = _nll_sc(input_tensor.T, target_tensor.astype(jnp.int32))
    return out[L - 1] + out[NC * L - 1]
